# Initial kernel scaffold; baseline (speedup 1.0000x reference)
import jax
import jax.numpy as jnp
from jax.experimental import pallas as pl


def kernel(images, bboxes, preds):
    raise NotImplementedError("write your pallas kernel here")


# speedup vs baseline: 9.4210x; 9.4210x over previous
import jax
import jax.numpy as jnp
from jax import lax
from jax.experimental import pallas as pl
from jax.experimental.pallas import tpu as pltpu

_LAMBDA_BBOX = 5.0
_IOU_THR = 0.1
_NEG = -1e30
_BIG_I = 2 ** 30


def _make_body(n_real):
  def _body(pch_ref, gt_ref, out_ref, iou_ref, acc_ref):
    b = pl.program_id(0)
    nb = pl.num_programs(0)
    R, C = iou_ref.shape[1], iou_ref.shape[2]
    G = iou_ref.shape[0]

    cx = pch_ref[0, 0]
    cy = pch_ref[0, 1]
    pw = pch_ref[0, 2]
    ph = pch_ref[0, 3]
    pc = pch_ref[0, 4]
    x1 = cx - pw / 2
    y1 = cy - ph / 2
    x2 = cx + pw / 2
    y2 = cy + ph / 2
    area_p = (x2 - x1) * (y2 - y1)

    flat_p = (lax.broadcasted_iota(jnp.int32, (R, C), 0) * C
              + lax.broadcasted_iota(jnp.int32, (R, C), 1))
    flat_s = (lax.broadcasted_iota(jnp.int32, (8, 128), 0) * 128
              + lax.broadcasted_iota(jnp.int32, (8, 128), 1))

    def gt_xyxy(j):
        gx = gt_ref[0, j, 0] / 512.0
        gy = gt_ref[0, j, 1] / 512.0
        gw = gt_ref[0, j, 2] / 512.0
        gh = gt_ref[0, j, 3] / 512.0
        gx1 = gx - gw / 2
        gy1 = gy - gh / 2
        gx2 = gx + gw / 2
        gy2 = gy + gh / 2
        return gx, gy, gw, gh, gx1, gy1, gx2, gy2

    def iou_col(j):
        _, _, _, _, gx1, gy1, gx2, gy2 = gt_xyxy(j)
        ga = (gx2 - gx1) * (gy2 - gy1)
        ltx = jnp.maximum(x1, gx1)
        lty = jnp.maximum(y1, gy1)
        rbx = jnp.minimum(x2, gx2)
        rby = jnp.minimum(y2, gy2)
        iw = jnp.clip(rbx - ltx, 0.0, None)
        ih = jnp.clip(rby - lty, 0.0, None)
        inter = iw * ih
        union = area_p + ga - inter
        return inter / jnp.maximum(union, 1e-9)

    def l1(j, colmax):
        col = iou_col(j)
        iou_ref[pl.ds(j, 1)] = col[None]
        m = jnp.max(col)
        return jnp.where(flat_s == j, m, colmax)

    colmax0 = jnp.full((8, 128), _NEG, jnp.float32)
    colmax = lax.fori_loop(0, G, l1, colmax0)

    def l2(k, st):
        colmax, used, bacc, cacc, nm = st
        m1 = jnp.max(colmax)
        jj = jnp.min(jnp.where(colmax == m1, flat_s, _BIG_I))
        col = iou_ref[pl.ds(jj, 1)][0]
        masked = jnp.where(used > 0.0, -1.0, col)
        m2 = jnp.max(masked)
        ii = jnp.min(jnp.where(masked == m2, flat_p, _BIG_I))
        ok = m2 >= _IOU_THR
        okf = jnp.where(ok, 1.0, 0.0)
        oh = flat_p == ii
        used = jnp.where(jnp.logical_and(oh, ok), 1.0, used)

        def gat(a):
            return jnp.sum(jnp.where(oh, a, 0.0))

        pcx_, pcy_, pw_, ph_, pcf_ = gat(cx), gat(cy), gat(pw), gat(ph), gat(pc)
        gx, gy, gw, gh, _, _, _, _ = gt_xyxy(jj)
        s = jnp.float32(0.0)
        for p_, g_ in ((pcx_, gx), (pcy_, gy), (pw_, gw), (ph_, gh)):
            d = jnp.abs(p_ - g_)
            s = s + jnp.where(d < 1.0, 0.5 * d * d, d - 0.5)
        bacc = bacc + okf * s
        logp = jnp.maximum(jnp.log(pcf_), -100.0)
        log1p_ = jnp.maximum(jnp.log(1.0 - pcf_), -100.0)
        cacc = cacc + okf * (log1p_ - logp)
        nm = nm + okf
        colmax = jnp.where(flat_s == jj, _NEG, colmax)
        return (colmax, used, bacc, cacc, nm)

    used0 = jnp.zeros((R, C), jnp.float32)
    z = jnp.float32(0.0)
    _, _, bacc, cacc, nm = lax.fori_loop(0, G, l2, (colmax, used0, z, z, z))

    base = jnp.sum(-jnp.maximum(jnp.log(1.0 - pc), -100.0))
    conf_b = base + cacc

    @pl.when(b == 0)
    def _():
        acc_ref[0] = 0.0
        acc_ref[1] = 0.0
        acc_ref[2] = 0.0

    acc_ref[0] = acc_ref[0] + bacc
    acc_ref[1] = acc_ref[1] + conf_b
    acc_ref[2] = acc_ref[2] + nm

    @pl.when(b == nb - 1)
    def _():
        tb = acc_ref[0]
        tcf = acc_ref[1]
        tm = acc_ref[2]
        nboxes = jnp.float32(nb * G)
        total_conf = tcf / jnp.float32(nb * n_real)
        has = tm > 0.0
        total_bbox = jnp.where(has, tb / jnp.maximum(tm, 1.0), 0.0)
        gap = jnp.where(has, (1.0 - tm / nboxes) * 2.0, 3.0)
        loss = _LAMBDA_BBOX * total_bbox + total_conf + gap
        rate = tm / nboxes
        o = jnp.where(flat_s == 0, loss,
            jnp.where(flat_s == 1, total_bbox,
            jnp.where(flat_s == 2, total_conf,
            jnp.where(flat_s == 3, gap,
            jnp.where(flat_s == 4, rate, 0.0)))))
        out_ref[...] = o

  return _body


def kernel(images, bboxes, preds):
    B, N, _ = preds.shape
    G = bboxes.shape[1]
    C = 128
    NPAD = ((N + 1023) // 1024) * 1024
    R = NPAD // C
    preds_p = jnp.pad(preds, ((0, 0), (0, NPAD - N), (0, 0)))
    pch = preds_p.transpose(0, 2, 1).reshape(B, 5, R, C)

    out = pl.pallas_call(
        _make_body(N),
        grid=(B,),
        in_specs=[
            pl.BlockSpec((1, 5, R, C), lambda i: (i, 0, 0, 0)),
            pl.BlockSpec((1, G, 4), lambda i: (i, 0, 0),
                         memory_space=pltpu.SMEM),
        ],
        out_specs=pl.BlockSpec((8, 128), lambda i: (0, 0)),
        out_shape=jax.ShapeDtypeStruct((8, 128), jnp.float32),
        scratch_shapes=[
            pltpu.VMEM((G, R, C), jnp.float32),
            pltpu.SMEM((3,), jnp.float32),
        ],
    )(pch, bboxes)
    return (out[0, 0], out[0, 1], out[0, 2], out[0, 3], out[0, 4])


# speedup vs baseline: 13.4704x; 1.4298x over previous
import jax
import jax.numpy as jnp
from jax import lax
from jax.experimental import pallas as pl
from jax.experimental.pallas import tpu as pltpu

_LAMBDA_BBOX = 5.0
_IOU_THR = 0.1
_NEG = -1e30
_BIG_I = 2 ** 30


def _make_body(n_real, n_batch, bpg):
  def _body(pch_ref, gt_ref, out_ref, iou_ref, used_ref, order_ref, acc_ref):
    gstep = pl.program_id(0)
    ngrid = pl.num_programs(0)
    G = iou_ref.shape[1]
    R, C = iou_ref.shape[2], iou_ref.shape[3]

    flat_p = (lax.broadcasted_iota(jnp.int32, (R, C), 0) * C
              + lax.broadcasted_iota(jnp.int32, (R, C), 1))
    flat_s = (lax.broadcasted_iota(jnp.int32, (8, 128), 0) * 128
              + lax.broadcasted_iota(jnp.int32, (8, 128), 1))
    lane = lax.broadcasted_iota(jnp.int32, (1, 128), 1)

    geom = []
    for bi in range(bpg):
      cx = pch_ref[bi, 0]
      cy = pch_ref[bi, 1]
      pw = pch_ref[bi, 2]
      ph = pch_ref[bi, 3]
      x1 = cx - pw / 2
      y1 = cy - ph / 2
      x2 = cx + pw / 2
      y2 = cy + ph / 2
      area_p = (x2 - x1) * (y2 - y1)
      geom.append((x1, y1, x2, y2, area_p))

    def gt_xyxy(bi, j):
      gx = gt_ref[bi, j, 0] / 512.0
      gy = gt_ref[bi, j, 1] / 512.0
      gw = gt_ref[bi, j, 2] / 512.0
      gh = gt_ref[bi, j, 3] / 512.0
      gx1 = gx - gw / 2
      gy1 = gy - gh / 2
      gx2 = gx + gw / 2
      gy2 = gy + gh / 2
      return gx, gy, gw, gh, gx1, gy1, gx2, gy2

    def iou_col(bi, j):
      _, _, _, _, gx1, gy1, gx2, gy2 = gt_xyxy(bi, j)
      x1, y1, x2, y2, area_p = geom[bi]
      ga = (gx2 - gx1) * (gy2 - gy1)
      ltx = jnp.maximum(x1, gx1)
      lty = jnp.maximum(y1, gy1)
      rbx = jnp.minimum(x2, gx2)
      rby = jnp.minimum(y2, gy2)
      iw = jnp.clip(rbx - ltx, 0.0, None)
      ih = jnp.clip(rby - lty, 0.0, None)
      inter = iw * ih
      union = area_p + ga - inter
      return inter / jnp.maximum(union, 1e-9)

    def l1(j, cms):
      out = []
      for bi in range(bpg):
        col = iou_col(bi, j)
        iou_ref[bi, pl.ds(j, 1)] = col[None]
        m = jnp.max(col)
        out.append(jnp.where(flat_s == j, m, cms[bi]))
      return tuple(out)

    cm0 = jnp.full((8, 128), _NEG, jnp.float32)
    cms = lax.fori_loop(0, G, l1, (cm0,) * bpg)

    def l2(k, cms):
      out = []
      for bi in range(bpg):
        m1 = jnp.max(cms[bi])
        jj = jnp.min(jnp.where(cms[bi] == m1, flat_s, _BIG_I))
        order_ref[bi, k] = jj
        out.append(jnp.where(flat_s == jj, _NEG, cms[bi]))
      return tuple(out)

    lax.fori_loop(0, G, l2, cms)

    used_ref[...] = jnp.zeros(used_ref.shape, jnp.float32)

    def l3(k, st):
      st = list(st)
      for bi in range(bpg):
        bacc, cacc, nm = st[3 * bi:3 * bi + 3]
        jj = order_ref[bi, k]
        col = iou_ref[bi, pl.ds(jj, 1)][0]
        masked = jnp.where(used_ref[bi] > 0.0, -1.0, col)
        m2 = jnp.max(masked)
        ii = jnp.min(jnp.where(masked == m2, flat_p, _BIG_I))
        ok = m2 >= _IOU_THR
        okf = jnp.where(ok, 1.0, 0.0)
        r_i = ii // C
        c_i = ii - r_i * C
        hit = lane == c_i
        row = used_ref[bi, pl.ds(r_i, 1), :]
        used_ref[bi, pl.ds(r_i, 1), :] = jnp.where(
            jnp.logical_and(hit, ok), 1.0, row)

        def gat(ch):
          rv = pch_ref[bi, ch, pl.ds(r_i, 1), :]
          return jnp.sum(jnp.where(hit, rv, 0.0))

        pcx_, pcy_, pw_, ph_, pcf_ = gat(0), gat(1), gat(2), gat(3), gat(4)
        gx, gy, gw, gh, _, _, _, _ = gt_xyxy(bi, jj)
        s = jnp.float32(0.0)
        for p_, g_ in ((pcx_, gx), (pcy_, gy), (pw_, gw), (ph_, gh)):
          d = jnp.abs(p_ - g_)
          s = s + jnp.where(d < 1.0, 0.5 * d * d, d - 0.5)
        logp = jnp.maximum(jnp.log(pcf_), -100.0)
        log1p_ = jnp.maximum(jnp.log(1.0 - pcf_), -100.0)
        st[3 * bi] = bacc + okf * s
        st[3 * bi + 1] = cacc + okf * (log1p_ - logp)
        st[3 * bi + 2] = nm + okf
      return tuple(st)

    z = jnp.float32(0.0)
    st = lax.fori_loop(0, G, l3, (z,) * (3 * bpg))

    bacc_t = jnp.float32(0.0)
    cacc_t = jnp.float32(0.0)
    nm_t = jnp.float32(0.0)
    for bi in range(bpg):
      pc = pch_ref[bi, 4]
      base = jnp.sum(-jnp.maximum(jnp.log(1.0 - pc), -100.0))
      bacc_t = bacc_t + st[3 * bi]
      cacc_t = cacc_t + base + st[3 * bi + 1]
      nm_t = nm_t + st[3 * bi + 2]

    @pl.when(gstep == 0)
    def _():
      acc_ref[0] = 0.0
      acc_ref[1] = 0.0
      acc_ref[2] = 0.0

    acc_ref[0] = acc_ref[0] + bacc_t
    acc_ref[1] = acc_ref[1] + cacc_t
    acc_ref[2] = acc_ref[2] + nm_t

    @pl.when(gstep == ngrid - 1)
    def _():
      tb = acc_ref[0]
      tcf = acc_ref[1]
      tm = acc_ref[2]
      nboxes = jnp.float32(n_batch * G)
      total_conf = tcf / jnp.float32(n_batch * n_real)
      has = tm > 0.0
      total_bbox = jnp.where(has, tb / jnp.maximum(tm, 1.0), 0.0)
      gap = jnp.where(has, (1.0 - tm / nboxes) * 2.0, 3.0)
      loss = _LAMBDA_BBOX * total_bbox + total_conf + gap
      rate = tm / nboxes
      o = jnp.where(flat_s == 0, loss,
          jnp.where(flat_s == 1, total_bbox,
          jnp.where(flat_s == 2, total_conf,
          jnp.where(flat_s == 3, gap,
          jnp.where(flat_s == 4, rate, 0.0)))))
      out_ref[...] = o

  return _body


def kernel(images, bboxes, preds):
  B, N, _ = preds.shape
  G = bboxes.shape[1]
  C = 128
  NPAD = ((N + 1023) // 1024) * 1024
  R = NPAD // C
  BPG = 4 if B % 4 == 0 else 1
  preds_p = jnp.pad(preds, ((0, 0), (0, NPAD - N), (0, 0)))
  pch = preds_p.transpose(0, 2, 1).reshape(B, 5, R, C)

  out = pl.pallas_call(
      _make_body(N, B, BPG),
      grid=(B // BPG,),
      in_specs=[
          pl.BlockSpec((BPG, 5, R, C), lambda i: (i, 0, 0, 0)),
          pl.BlockSpec((BPG, G, 4), lambda i: (i, 0, 0),
                       memory_space=pltpu.SMEM),
      ],
      out_specs=pl.BlockSpec((8, 128), lambda i: (0, 0)),
      out_shape=jax.ShapeDtypeStruct((8, 128), jnp.float32),
      scratch_shapes=[
          pltpu.VMEM((BPG, G, R, C), jnp.float32),
          pltpu.VMEM((BPG, R, C), jnp.float32),
          pltpu.SMEM((BPG, 128), jnp.int32),
          pltpu.SMEM((3,), jnp.float32),
      ],
  )(pch, bboxes)
  return (out[0, 0], out[0, 1], out[0, 2], out[0, 3], out[0, 4])


# speedup vs baseline: 14.0562x; 1.0435x over previous
import jax
import jax.numpy as jnp
from jax import lax
from jax.experimental import pallas as pl
from jax.experimental.pallas import tpu as pltpu

_LAMBDA_BBOX = 5.0
_IOU_THR = 0.1
_NEG = -1e30
_BIG_I = 2 ** 30


def _make_body(n_real, n_batch, bpg):
  def _body(pch_ref, gt_ref, out_ref, iou_ref, used_ref, order_ref, acc_ref):
    gstep = pl.program_id(0)
    ngrid = pl.num_programs(0)
    G = iou_ref.shape[1]
    R, C = iou_ref.shape[2], iou_ref.shape[3]

    flat_p = (lax.broadcasted_iota(jnp.int32, (R, C), 0) * C
              + lax.broadcasted_iota(jnp.int32, (R, C), 1))
    flat_s = (lax.broadcasted_iota(jnp.int32, (8, 128), 0) * 128
              + lax.broadcasted_iota(jnp.int32, (8, 128), 1))
    lane = lax.broadcasted_iota(jnp.int32, (1, 128), 1)

    geom = []
    for bi in range(bpg):
      cx = pch_ref[bi, 0]
      cy = pch_ref[bi, 1]
      pw = pch_ref[bi, 2]
      ph = pch_ref[bi, 3]
      x1 = cx - pw / 2
      y1 = cy - ph / 2
      x2 = cx + pw / 2
      y2 = cy + ph / 2
      area_p = (x2 - x1) * (y2 - y1)
      geom.append((x1, y1, x2, y2, area_p))

    def gt_xyxy(bi, j):
      gx = gt_ref[bi, j, 0] / 512.0
      gy = gt_ref[bi, j, 1] / 512.0
      gw = gt_ref[bi, j, 2] / 512.0
      gh = gt_ref[bi, j, 3] / 512.0
      gx1 = gx - gw / 2
      gy1 = gy - gh / 2
      gx2 = gx + gw / 2
      gy2 = gy + gh / 2
      return gx, gy, gw, gh, gx1, gy1, gx2, gy2

    def iou_col(bi, j):
      _, _, _, _, gx1, gy1, gx2, gy2 = gt_xyxy(bi, j)
      x1, y1, x2, y2, area_p = geom[bi]
      ga = (gx2 - gx1) * (gy2 - gy1)
      ltx = jnp.maximum(x1, gx1)
      lty = jnp.maximum(y1, gy1)
      rbx = jnp.minimum(x2, gx2)
      rby = jnp.minimum(y2, gy2)
      iw = jnp.clip(rbx - ltx, 0.0, None)
      ih = jnp.clip(rby - lty, 0.0, None)
      inter = iw * ih
      union = area_p + ga - inter
      return inter / jnp.maximum(union, 1e-9)

    def l1(t, cms):
      out = list(cms)
      for u in range(2):
        j = t * 2 + u
        for bi in range(bpg):
          col = iou_col(bi, j)
          iou_ref[bi, pl.ds(j, 1)] = col[None]
          m = jnp.max(col)
          out[bi] = jnp.where(flat_s == j, m, out[bi])
      return tuple(out)

    cm0 = jnp.full((8, 128), _NEG, jnp.float32)
    cms = lax.fori_loop(0, G // 2, l1, (cm0,) * bpg)

    def l2(k, cms):
      out = []
      for bi in range(bpg):
        m1 = jnp.max(cms[bi])
        jj = jnp.min(jnp.where(cms[bi] == m1, flat_s, _BIG_I))
        order_ref[bi, k] = jj
        out.append(jnp.where(flat_s == jj, _NEG, cms[bi]))
      return tuple(out)

    lax.fori_loop(0, G, l2, cms)

    used_ref[...] = jnp.zeros(used_ref.shape, jnp.float32)

    def l3(t, st):
      st = list(st)
      for u in range(2):
        k = t * 2 + u
        for bi in range(bpg):
          bacc, cacc, nm = st[3 * bi:3 * bi + 3]
          jj = order_ref[bi, k]
          col = iou_ref[bi, pl.ds(jj, 1)][0]
          masked = col - used_ref[bi]
          m2 = jnp.max(masked)
          ii = jnp.min(jnp.where(masked == m2, flat_p, _BIG_I))
          ok = m2 >= _IOU_THR
          okf = jnp.where(ok, 1.0, 0.0)
          r_i = ii // C
          c_i = ii - r_i * C
          hit = lane == c_i
          row = used_ref[bi, pl.ds(r_i, 1), :]
          used_ref[bi, pl.ds(r_i, 1), :] = jnp.where(
              jnp.logical_and(hit, ok), 2.0, row)

          def gat(ch):
            rv = pch_ref[bi, ch, pl.ds(r_i, 1), :]
            return jnp.sum(jnp.where(hit, rv, 0.0))

          pcx_, pcy_, pw_, ph_, pcf_ = gat(0), gat(1), gat(2), gat(3), gat(4)
          gx, gy, gw, gh, _, _, _, _ = gt_xyxy(bi, jj)
          s = jnp.float32(0.0)
          for p_, g_ in ((pcx_, gx), (pcy_, gy), (pw_, gw), (ph_, gh)):
            d = jnp.abs(p_ - g_)
            s = s + jnp.where(d < 1.0, 0.5 * d * d, d - 0.5)
          logp = jnp.maximum(jnp.log(pcf_), -100.0)
          log1p_ = jnp.maximum(jnp.log(1.0 - pcf_), -100.0)
          st[3 * bi] = bacc + okf * s
          st[3 * bi + 1] = cacc + okf * (log1p_ - logp)
          st[3 * bi + 2] = nm + okf
      return tuple(st)

    z = jnp.float32(0.0)
    st = lax.fori_loop(0, G // 2, l3, (z,) * (3 * bpg))

    bacc_t = jnp.float32(0.0)
    cacc_t = jnp.float32(0.0)
    nm_t = jnp.float32(0.0)
    for bi in range(bpg):
      pc = pch_ref[bi, 4]
      base = jnp.sum(-jnp.maximum(jnp.log(1.0 - pc), -100.0))
      bacc_t = bacc_t + st[3 * bi]
      cacc_t = cacc_t + base + st[3 * bi + 1]
      nm_t = nm_t + st[3 * bi + 2]

    @pl.when(gstep == 0)
    def _():
      acc_ref[0] = 0.0
      acc_ref[1] = 0.0
      acc_ref[2] = 0.0

    acc_ref[0] = acc_ref[0] + bacc_t
    acc_ref[1] = acc_ref[1] + cacc_t
    acc_ref[2] = acc_ref[2] + nm_t

    @pl.when(gstep == ngrid - 1)
    def _():
      tb = acc_ref[0]
      tcf = acc_ref[1]
      tm = acc_ref[2]
      nboxes = jnp.float32(n_batch * G)
      total_conf = tcf / jnp.float32(n_batch * n_real)
      has = tm > 0.0
      total_bbox = jnp.where(has, tb / jnp.maximum(tm, 1.0), 0.0)
      gap = jnp.where(has, (1.0 - tm / nboxes) * 2.0, 3.0)
      loss = _LAMBDA_BBOX * total_bbox + total_conf + gap
      rate = tm / nboxes
      o = jnp.where(flat_s == 0, loss,
          jnp.where(flat_s == 1, total_bbox,
          jnp.where(flat_s == 2, total_conf,
          jnp.where(flat_s == 3, gap,
          jnp.where(flat_s == 4, rate, 0.0)))))
      out_ref[...] = o

  return _body


def kernel(images, bboxes, preds):
  B, N, _ = preds.shape
  G = bboxes.shape[1]
  C = 128
  NPAD = ((N + 1023) // 1024) * 1024
  R = NPAD // C
  BPG = 4 if B % 4 == 0 else 1
  preds_p = jnp.pad(preds, ((0, 0), (0, NPAD - N), (0, 0)))
  pch = preds_p.transpose(0, 2, 1).reshape(B, 5, R, C)

  out = pl.pallas_call(
      _make_body(N, B, BPG),
      grid=(B // BPG,),
      in_specs=[
          pl.BlockSpec((BPG, 5, R, C), lambda i: (i, 0, 0, 0)),
          pl.BlockSpec((BPG, G, 4), lambda i: (i, 0, 0),
                       memory_space=pltpu.SMEM),
      ],
      out_specs=pl.BlockSpec((8, 128), lambda i: (0, 0)),
      out_shape=jax.ShapeDtypeStruct((8, 128), jnp.float32),
      scratch_shapes=[
          pltpu.VMEM((BPG, G, R, C), jnp.float32),
          pltpu.VMEM((BPG, R, C), jnp.float32),
          pltpu.SMEM((BPG, 128), jnp.int32),
          pltpu.SMEM((3,), jnp.float32),
      ],
  )(pch, bboxes)
  return (out[0, 0], out[0, 1], out[0, 2], out[0, 3], out[0, 4])


# speedup vs baseline: 14.0586x; 1.0002x over previous
import jax
import jax.numpy as jnp
from jax import lax
from jax.experimental import pallas as pl
from jax.experimental.pallas import tpu as pltpu

_LAMBDA_BBOX = 5.0
_IOU_THR = 0.1
_NEG = -1e30
_BIG_I = 2 ** 30


def _make_body(n_real, n_batch, bpg):
  def _body(pch_ref, gt_ref, out_ref, *scr):
    iou_refs = scr[0:bpg]
    used_refs = scr[bpg:2 * bpg]
    order_refs = scr[2 * bpg:3 * bpg]
    acc_ref = scr[3 * bpg]
    gstep = pl.program_id(0)
    ngrid = pl.num_programs(0)
    G = iou_refs[0].shape[0]
    R, C = iou_refs[0].shape[1], iou_refs[0].shape[2]

    flat_p = (lax.broadcasted_iota(jnp.int32, (R, C), 0) * C
              + lax.broadcasted_iota(jnp.int32, (R, C), 1))
    flat_s = (lax.broadcasted_iota(jnp.int32, (8, 128), 0) * 128
              + lax.broadcasted_iota(jnp.int32, (8, 128), 1))
    lane = lax.broadcasted_iota(jnp.int32, (1, 128), 1)

    geom = []
    for bi in range(bpg):
      cx = pch_ref[bi, 0]
      cy = pch_ref[bi, 1]
      pw = pch_ref[bi, 2]
      ph = pch_ref[bi, 3]
      x1 = cx - pw / 2
      y1 = cy - ph / 2
      x2 = cx + pw / 2
      y2 = cy + ph / 2
      area_p = (x2 - x1) * (y2 - y1)
      geom.append((x1, y1, x2, y2, area_p))

    def gt_xyxy(bi, j):
      gx = gt_ref[bi, j, 0] / 512.0
      gy = gt_ref[bi, j, 1] / 512.0
      gw = gt_ref[bi, j, 2] / 512.0
      gh = gt_ref[bi, j, 3] / 512.0
      gx1 = gx - gw / 2
      gy1 = gy - gh / 2
      gx2 = gx + gw / 2
      gy2 = gy + gh / 2
      return gx, gy, gw, gh, gx1, gy1, gx2, gy2

    def iou_col(bi, j):
      _, _, _, _, gx1, gy1, gx2, gy2 = gt_xyxy(bi, j)
      x1, y1, x2, y2, area_p = geom[bi]
      ga = (gx2 - gx1) * (gy2 - gy1)
      ltx = jnp.maximum(x1, gx1)
      lty = jnp.maximum(y1, gy1)
      rbx = jnp.minimum(x2, gx2)
      rby = jnp.minimum(y2, gy2)
      iw = jnp.clip(rbx - ltx, 0.0, None)
      ih = jnp.clip(rby - lty, 0.0, None)
      inter = iw * ih
      union = area_p + ga - inter
      return inter / jnp.maximum(union, 1e-9)

    def l1(t, cms):
      out = list(cms)
      for u in range(2):
        j = t * 2 + u
        for bi in range(bpg):
          col = iou_col(bi, j)
          iou_refs[bi][pl.ds(j, 1)] = col[None]
          m = jnp.max(col)
          out[bi] = jnp.where(flat_s == j, m, out[bi])
      return tuple(out)

    cm0 = jnp.full((8, 128), _NEG, jnp.float32)
    cms = lax.fori_loop(0, G // 2, l1, (cm0,) * bpg)

    def l2(k, cms):
      out = []
      for bi in range(bpg):
        m1 = jnp.max(cms[bi])
        jj = jnp.min(jnp.where(cms[bi] == m1, flat_s, _BIG_I))
        order_refs[bi][k] = jj
        out.append(jnp.where(flat_s == jj, _NEG, cms[bi]))
      return tuple(out)

    lax.fori_loop(0, G, l2, cms)

    for bi in range(bpg):
      used_refs[bi][...] = jnp.zeros((R, C), jnp.float32)

    def l3(t, st):
      st = list(st)
      for u in range(2):
        k = t * 2 + u
        for bi in range(bpg):
          bacc, cacc, nm = st[3 * bi:3 * bi + 3]
          jj = order_refs[bi][k]
          col = iou_refs[bi][pl.ds(jj, 1)][0]
          masked = col - used_refs[bi][...]
          m2 = jnp.max(masked)
          ii = jnp.min(jnp.where(masked == m2, flat_p, _BIG_I))
          ok = m2 >= _IOU_THR
          okf = jnp.where(ok, 1.0, 0.0)
          r_i = ii // C
          c_i = ii - r_i * C
          hit = lane == c_i
          row = used_refs[bi][pl.ds(r_i, 1), :]
          used_refs[bi][pl.ds(r_i, 1), :] = jnp.where(
              jnp.logical_and(hit, ok), 2.0, row)

          def gat(ch):
            rv = pch_ref[bi, ch, pl.ds(r_i, 1), :]
            return jnp.sum(jnp.where(hit, rv, 0.0))

          pcx_, pcy_, pw_, ph_, pcf_ = gat(0), gat(1), gat(2), gat(3), gat(4)
          gx, gy, gw, gh, _, _, _, _ = gt_xyxy(bi, jj)
          s = jnp.float32(0.0)
          for p_, g_ in ((pcx_, gx), (pcy_, gy), (pw_, gw), (ph_, gh)):
            d = jnp.abs(p_ - g_)
            s = s + jnp.where(d < 1.0, 0.5 * d * d, d - 0.5)
          logp = jnp.maximum(jnp.log(pcf_), -100.0)
          log1p_ = jnp.maximum(jnp.log(1.0 - pcf_), -100.0)
          st[3 * bi] = bacc + okf * s
          st[3 * bi + 1] = cacc + okf * (log1p_ - logp)
          st[3 * bi + 2] = nm + okf
      return tuple(st)

    z = jnp.float32(0.0)
    st = lax.fori_loop(0, G // 2, l3, (z,) * (3 * bpg))

    bacc_t = jnp.float32(0.0)
    cacc_t = jnp.float32(0.0)
    nm_t = jnp.float32(0.0)
    for bi in range(bpg):
      pc = pch_ref[bi, 4]
      base = jnp.sum(-jnp.maximum(jnp.log(1.0 - pc), -100.0))
      bacc_t = bacc_t + st[3 * bi]
      cacc_t = cacc_t + base + st[3 * bi + 1]
      nm_t = nm_t + st[3 * bi + 2]

    @pl.when(gstep == 0)
    def _():
      acc_ref[0] = 0.0
      acc_ref[1] = 0.0
      acc_ref[2] = 0.0

    acc_ref[0] = acc_ref[0] + bacc_t
    acc_ref[1] = acc_ref[1] + cacc_t
    acc_ref[2] = acc_ref[2] + nm_t

    @pl.when(gstep == ngrid - 1)
    def _():
      tb = acc_ref[0]
      tcf = acc_ref[1]
      tm = acc_ref[2]
      nboxes = jnp.float32(n_batch * G)
      total_conf = tcf / jnp.float32(n_batch * n_real)
      has = tm > 0.0
      total_bbox = jnp.where(has, tb / jnp.maximum(tm, 1.0), 0.0)
      gap = jnp.where(has, (1.0 - tm / nboxes) * 2.0, 3.0)
      loss = _LAMBDA_BBOX * total_bbox + total_conf + gap
      rate = tm / nboxes
      o = jnp.where(flat_s == 0, loss,
          jnp.where(flat_s == 1, total_bbox,
          jnp.where(flat_s == 2, total_conf,
          jnp.where(flat_s == 3, gap,
          jnp.where(flat_s == 4, rate, 0.0)))))
      out_ref[...] = o

  return _body


def kernel(images, bboxes, preds):
  B, N, _ = preds.shape
  G = bboxes.shape[1]
  C = 128
  NPAD = ((N + 1023) // 1024) * 1024
  R = NPAD // C
  BPG = 4 if B % 4 == 0 else 1
  preds_p = jnp.pad(preds, ((0, 0), (0, NPAD - N), (0, 0)))
  pch = preds_p.transpose(0, 2, 1).reshape(B, 5, R, C)

  out = pl.pallas_call(
      _make_body(N, B, BPG),
      grid=(B // BPG,),
      in_specs=[
          pl.BlockSpec((BPG, 5, R, C), lambda i: (i, 0, 0, 0)),
          pl.BlockSpec((BPG, G, 4), lambda i: (i, 0, 0),
                       memory_space=pltpu.SMEM),
      ],
      out_specs=pl.BlockSpec((8, 128), lambda i: (0, 0)),
      out_shape=jax.ShapeDtypeStruct((8, 128), jnp.float32),
      scratch_shapes=(
          [pltpu.VMEM((G, R, C), jnp.float32) for _ in range(BPG)]
          + [pltpu.VMEM((R, C), jnp.float32) for _ in range(BPG)]
          + [pltpu.SMEM((128,), jnp.int32) for _ in range(BPG)]
          + [pltpu.SMEM((3,), jnp.float32)]
      ),
  )(pch, bboxes)
  return (out[0, 0], out[0, 1], out[0, 2], out[0, 3], out[0, 4])


# speedup vs baseline: 16.9167x; 1.2033x over previous
import jax
import jax.numpy as jnp
from jax import lax
from jax.experimental import pallas as pl
from jax.experimental.pallas import tpu as pltpu

_LAMBDA_BBOX = 5.0
_IOU_THR = 0.1
_NEG = -1e30
_BIG_I = 2 ** 30


def _make_body(n_real, n_batch, bpg):
  def _body(pch_ref, gt_ref, out_ref, *scr):
    iou_refs = scr[0:bpg]
    used_refs = scr[bpg:2 * bpg]
    order_refs = scr[2 * bpg:3 * bpg]
    acc_ref = scr[3 * bpg]
    gstep = pl.program_id(0)
    ngrid = pl.num_programs(0)
    G = iou_refs[0].shape[0]
    R, C = iou_refs[0].shape[1], iou_refs[0].shape[2]

    flat_p = (lax.broadcasted_iota(jnp.int32, (R, C), 0) * C
              + lax.broadcasted_iota(jnp.int32, (R, C), 1))
    flat_s = (lax.broadcasted_iota(jnp.int32, (8, 128), 0) * 128
              + lax.broadcasted_iota(jnp.int32, (8, 128), 1))
    lane = lax.broadcasted_iota(jnp.int32, (1, 128), 1)

    geom = []
    for bi in range(bpg):
      cx = pch_ref[bi, 0]
      cy = pch_ref[bi, 1]
      pw = pch_ref[bi, 2]
      ph = pch_ref[bi, 3]
      x1 = cx - pw / 2
      y1 = cy - ph / 2
      x2 = cx + pw / 2
      y2 = cy + ph / 2
      area_p = (x2 - x1) * (y2 - y1)
      geom.append((x1, y1, x2, y2, area_p))

    def gt_xyxy(bi, j):
      gx = gt_ref[bi, j, 0] / 512.0
      gy = gt_ref[bi, j, 1] / 512.0
      gw = gt_ref[bi, j, 2] / 512.0
      gh = gt_ref[bi, j, 3] / 512.0
      gx1 = gx - gw / 2
      gy1 = gy - gh / 2
      gx2 = gx + gw / 2
      gy2 = gy + gh / 2
      return gx, gy, gw, gh, gx1, gy1, gx2, gy2

    def iou_col(bi, j):
      _, _, _, _, gx1, gy1, gx2, gy2 = gt_xyxy(bi, j)
      x1, y1, x2, y2, area_p = geom[bi]
      ga = (gx2 - gx1) * (gy2 - gy1)
      ltx = jnp.maximum(x1, gx1)
      lty = jnp.maximum(y1, gy1)
      rbx = jnp.minimum(x2, gx2)
      rby = jnp.minimum(y2, gy2)
      iw = jnp.clip(rbx - ltx, 0.0, None)
      ih = jnp.clip(rby - lty, 0.0, None)
      inter = iw * ih
      union = area_p + ga - inter
      return inter / jnp.maximum(union, 1e-9)

    def l1(t, cms):
      out = list(cms)
      for u in range(2):
        j = t * 2 + u
        for bi in range(bpg):
          col = iou_col(bi, j)
          iou_refs[bi][pl.ds(j, 1)] = col[None]
          m = jnp.max(col, axis=(0, 1), keepdims=True)
          out[bi] = jnp.where(flat_s == j, m, out[bi])
      return tuple(out)

    cm0 = jnp.full((8, 128), _NEG, jnp.float32)
    cms = lax.fori_loop(0, G // 2, l1, (cm0,) * bpg)

    def l2(k, cms):
      out = []
      for bi in range(bpg):
        m1 = jnp.max(cms[bi])
        jj = jnp.min(jnp.where(cms[bi] == m1, flat_s, _BIG_I))
        order_refs[bi][k] = jj
        out.append(jnp.where(flat_s == jj, _NEG, cms[bi]))
      return tuple(out)

    lax.fori_loop(0, G, l2, cms)

    for bi in range(bpg):
      used_refs[bi][...] = jnp.zeros((R, C), jnp.float32)

    RR = R // 8

    def fold8(x):
      return jnp.sum(x.reshape(RR, 8, C), axis=0)

    def l3(t, st):
      st = list(st)
      for u in range(2):
        k = t * 2 + u
        for bi in range(bpg):
          bacc_v, cacc_v, nm_v = st[3 * bi:3 * bi + 3]
          jj = order_refs[bi][k]
          col = iou_refs[bi][pl.ds(jj, 1)][0]
          us = used_refs[bi][...]
          masked = col - us
          m2 = jnp.max(masked, axis=(0, 1), keepdims=True)
          okv = m2 >= _IOU_THR
          eq = masked == m2
          ii = jnp.min(jnp.where(eq, flat_p, _BIG_I), axis=(0, 1),
                       keepdims=True)
          oh = flat_p == ii
          used_refs[bi][...] = jnp.where(
              jnp.logical_and(oh, okv), 2.0, us)
          oh8 = fold8(jnp.where(oh, 1.0, 0.0))
          okm = jnp.where(okv, oh8, 0.0)
          nm_v = nm_v + okm
          gx, gy, gw, gh, _, _, _, _ = gt_xyxy(bi, jj)
          el = None
          for ch, g_ in ((0, gx), (1, gy), (2, gw), (3, gh)):
            v8 = fold8(jnp.where(oh, pch_ref[bi, ch], 0.0))
            d = jnp.abs(v8 - g_)
            e = jnp.where(d < 1.0, 0.5 * d * d, d - 0.5)
            el = e if el is None else el + e
          bacc_v = bacc_v + okm * el
          pv = fold8(jnp.where(oh, pch_ref[bi, 4], 0.0))
          logp = jnp.maximum(jnp.log(pv), -100.0)
          log1p_ = jnp.maximum(jnp.log(1.0 - pv), -100.0)
          cacc_v = cacc_v + okm * (log1p_ - logp)
          st[3 * bi] = bacc_v
          st[3 * bi + 1] = cacc_v
          st[3 * bi + 2] = nm_v
      return tuple(st)

    zv = jnp.zeros((8, C), jnp.float32)
    st = lax.fori_loop(0, G // 2, l3, (zv,) * (3 * bpg))

    bacc_t = jnp.float32(0.0)
    cacc_t = jnp.float32(0.0)
    nm_t = jnp.float32(0.0)
    for bi in range(bpg):
      pc = pch_ref[bi, 4]
      base = jnp.sum(-jnp.maximum(jnp.log(1.0 - pc), -100.0))
      bacc_t = bacc_t + jnp.sum(st[3 * bi])
      cacc_t = cacc_t + base + jnp.sum(st[3 * bi + 1])
      nm_t = nm_t + jnp.sum(st[3 * bi + 2])

    @pl.when(gstep == 0)
    def _():
      acc_ref[0] = 0.0
      acc_ref[1] = 0.0
      acc_ref[2] = 0.0

    acc_ref[0] = acc_ref[0] + bacc_t
    acc_ref[1] = acc_ref[1] + cacc_t
    acc_ref[2] = acc_ref[2] + nm_t

    @pl.when(gstep == ngrid - 1)
    def _():
      tb = acc_ref[0]
      tcf = acc_ref[1]
      tm = acc_ref[2]
      nboxes = jnp.float32(n_batch * G)
      total_conf = tcf / jnp.float32(n_batch * n_real)
      has = tm > 0.0
      total_bbox = jnp.where(has, tb / jnp.maximum(tm, 1.0), 0.0)
      gap = jnp.where(has, (1.0 - tm / nboxes) * 2.0, 3.0)
      loss = _LAMBDA_BBOX * total_bbox + total_conf + gap
      rate = tm / nboxes
      o = jnp.where(flat_s == 0, loss,
          jnp.where(flat_s == 1, total_bbox,
          jnp.where(flat_s == 2, total_conf,
          jnp.where(flat_s == 3, gap,
          jnp.where(flat_s == 4, rate, 0.0)))))
      out_ref[...] = o

  return _body


def kernel(images, bboxes, preds):
  B, N, _ = preds.shape
  G = bboxes.shape[1]
  C = 128
  NPAD = ((N + 1023) // 1024) * 1024
  R = NPAD // C
  BPG = 4 if B % 4 == 0 else 1
  preds_p = jnp.pad(preds, ((0, 0), (0, NPAD - N), (0, 0)))
  pch = preds_p.transpose(0, 2, 1).reshape(B, 5, R, C)

  out = pl.pallas_call(
      _make_body(N, B, BPG),
      grid=(B // BPG,),
      in_specs=[
          pl.BlockSpec((BPG, 5, R, C), lambda i: (i, 0, 0, 0)),
          pl.BlockSpec((BPG, G, 4), lambda i: (i, 0, 0),
                       memory_space=pltpu.SMEM),
      ],
      out_specs=pl.BlockSpec((8, 128), lambda i: (0, 0)),
      out_shape=jax.ShapeDtypeStruct((8, 128), jnp.float32),
      scratch_shapes=(
          [pltpu.VMEM((G, R, C), jnp.float32) for _ in range(BPG)]
          + [pltpu.VMEM((R, C), jnp.float32) for _ in range(BPG)]
          + [pltpu.SMEM((128,), jnp.int32) for _ in range(BPG)]
          + [pltpu.SMEM((3,), jnp.float32)]
      ),
  )(pch, bboxes)
  return (out[0, 0], out[0, 1], out[0, 2], out[0, 3], out[0, 4])


# speedup vs baseline: 32.7127x; 1.9337x over previous
import jax
import jax.numpy as jnp
from jax import lax
from jax.experimental import pallas as pl
from jax.experimental.pallas import tpu as pltpu

_LAMBDA_BBOX = 5.0
_IOU_THR = 0.1
_NEG = -1e30
_BIG_I = 2 ** 30


def _make_body(n_real, n_batch, bpg):
  def _body(pch_ref, gt_ref, out_ref, *scr):
    iou_refs = scr[0:bpg]
    order_refs = scr[bpg:2 * bpg]
    acc_ref = scr[2 * bpg]
    gstep = pl.program_id(0)
    ngrid = pl.num_programs(0)
    G = iou_refs[0].shape[0]
    R, C = iou_refs[0].shape[1], iou_refs[0].shape[2]

    flat_p = (lax.broadcasted_iota(jnp.int32, (R, C), 0) * C
              + lax.broadcasted_iota(jnp.int32, (R, C), 1))
    flat_s = (lax.broadcasted_iota(jnp.int32, (8, 128), 0) * 128
              + lax.broadcasted_iota(jnp.int32, (8, 128), 1))
    lane = lax.broadcasted_iota(jnp.int32, (1, 128), 1)

    geom = []
    for bi in range(bpg):
      cx = pch_ref[bi, 0]
      cy = pch_ref[bi, 1]
      pw = pch_ref[bi, 2]
      ph = pch_ref[bi, 3]
      x1 = cx - pw / 2
      y1 = cy - ph / 2
      x2 = cx + pw / 2
      y2 = cy + ph / 2
      area_p = (x2 - x1) * (y2 - y1)
      geom.append((x1, y1, x2, y2, area_p))

    def gt_xyxy(bi, j):
      gx = gt_ref[bi, j, 0] / 512.0
      gy = gt_ref[bi, j, 1] / 512.0
      gw = gt_ref[bi, j, 2] / 512.0
      gh = gt_ref[bi, j, 3] / 512.0
      gx1 = gx - gw / 2
      gy1 = gy - gh / 2
      gx2 = gx + gw / 2
      gy2 = gy + gh / 2
      return gx, gy, gw, gh, gx1, gy1, gx2, gy2

    def iou_col(bi, j):
      _, _, _, _, gx1, gy1, gx2, gy2 = gt_xyxy(bi, j)
      x1, y1, x2, y2, area_p = geom[bi]
      ga = (gx2 - gx1) * (gy2 - gy1)
      ltx = jnp.maximum(x1, gx1)
      lty = jnp.maximum(y1, gy1)
      rbx = jnp.minimum(x2, gx2)
      rby = jnp.minimum(y2, gy2)
      iw = jnp.clip(rbx - ltx, 0.0, None)
      ih = jnp.clip(rby - lty, 0.0, None)
      inter = iw * ih
      union = area_p + ga - inter
      return inter / jnp.maximum(union, 1e-9)

    def l1(t, cms):
      out = list(cms)
      for u in range(2):
        j = t * 2 + u
        for bi in range(bpg):
          col = iou_col(bi, j)
          iou_refs[bi][pl.ds(j, 1)] = col[None]
          m = jnp.max(col, axis=(0, 1), keepdims=True)
          out[bi] = jnp.where(flat_s == j, m, out[bi])
      return tuple(out)

    cm0 = jnp.full((8, 128), _NEG, jnp.float32)
    cms = lax.fori_loop(0, G // 2, l1, (cm0,) * bpg)

    def l2(k, cms):
      m1 = [jnp.max(cms[bi], axis=(0, 1), keepdims=True)
            for bi in range(bpg)]
      jjv = [jnp.min(jnp.where(cms[bi] == m1[bi], flat_s, _BIG_I),
                     axis=(0, 1), keepdims=True) for bi in range(bpg)]
      for bi in range(bpg):
        order_refs[bi][k] = jjv[bi][0, 0]
      return tuple(jnp.where(flat_s == jjv[bi], _NEG, cms[bi])
                   for bi in range(bpg))

    lax.fori_loop(0, G, l2, cms)


    RR = R // 8

    def fold8(x):
      return jnp.sum(x.reshape(RR, 8, C), axis=0)

    def l3(t, st):
      st = list(st)
      for u in range(2):
        k = t * 2 + u
        used = [st[4 * bi] for bi in range(bpg)]
        jj = [order_refs[bi][k] for bi in range(bpg)]
        col = [iou_refs[bi][pl.ds(jj[bi], 1)][0] for bi in range(bpg)]
        masked = [col[bi] - used[bi] for bi in range(bpg)]
        m2 = [jnp.max(masked[bi], axis=(0, 1), keepdims=True)
              for bi in range(bpg)]
        okv = [m2[bi] >= _IOU_THR for bi in range(bpg)]
        eq = [masked[bi] == m2[bi] for bi in range(bpg)]
        ii = [jnp.min(jnp.where(eq[bi], flat_p, _BIG_I), axis=(0, 1),
                      keepdims=True) for bi in range(bpg)]
        oh = [flat_p == ii[bi] for bi in range(bpg)]
        for bi in range(bpg):
          st[4 * bi] = jnp.where(
              jnp.logical_and(oh[bi], okv[bi]), 2.0, used[bi])
        oh8 = [fold8(jnp.where(oh[bi], 1.0, 0.0)) for bi in range(bpg)]
        okm = [jnp.where(okv[bi], oh8[bi], 0.0) for bi in range(bpg)]
        for bi in range(bpg):
          bacc_v, cacc_v, nm_v = st[4 * bi + 1:4 * bi + 4]
          nm_v = nm_v + okm[bi]
          gx, gy, gw, gh, _, _, _, _ = gt_xyxy(bi, jj[bi])
          el = None
          for ch, g_ in ((0, gx), (1, gy), (2, gw), (3, gh)):
            v8 = fold8(jnp.where(oh[bi], pch_ref[bi, ch], 0.0))
            d = jnp.abs(v8 - g_)
            e = jnp.where(d < 1.0, 0.5 * d * d, d - 0.5)
            el = e if el is None else el + e
          bacc_v = bacc_v + okm[bi] * el
          pv = fold8(jnp.where(oh[bi], pch_ref[bi, 4], 0.0))
          logp = jnp.maximum(jnp.log(pv), -100.0)
          log1p_ = jnp.maximum(jnp.log(1.0 - pv), -100.0)
          cacc_v = cacc_v + okm[bi] * (log1p_ - logp)
          st[4 * bi + 1] = bacc_v
          st[4 * bi + 2] = cacc_v
          st[4 * bi + 3] = nm_v
      return tuple(st)

    zv = jnp.zeros((8, C), jnp.float32)
    zu = jnp.zeros((R, C), jnp.float32)
    st0 = []
    for bi in range(bpg):
      st0 += [zu, zv, zv, zv]
    st = lax.fori_loop(0, G // 2, l3, tuple(st0))

    bacc_t = jnp.float32(0.0)
    cacc_t = jnp.float32(0.0)
    nm_t = jnp.float32(0.0)
    for bi in range(bpg):
      pc = pch_ref[bi, 4]
      base = jnp.sum(-jnp.maximum(jnp.log(1.0 - pc), -100.0))
      bacc_t = bacc_t + jnp.sum(st[4 * bi + 1])
      cacc_t = cacc_t + base + jnp.sum(st[4 * bi + 2])
      nm_t = nm_t + jnp.sum(st[4 * bi + 3])

    @pl.when(gstep == 0)
    def _():
      acc_ref[0] = 0.0
      acc_ref[1] = 0.0
      acc_ref[2] = 0.0

    acc_ref[0] = acc_ref[0] + bacc_t
    acc_ref[1] = acc_ref[1] + cacc_t
    acc_ref[2] = acc_ref[2] + nm_t

    @pl.when(gstep == ngrid - 1)
    def _():
      tb = acc_ref[0]
      tcf = acc_ref[1]
      tm = acc_ref[2]
      nboxes = jnp.float32(n_batch * G)
      total_conf = tcf / jnp.float32(n_batch * n_real)
      has = tm > 0.0
      total_bbox = jnp.where(has, tb / jnp.maximum(tm, 1.0), 0.0)
      gap = jnp.where(has, (1.0 - tm / nboxes) * 2.0, 3.0)
      loss = _LAMBDA_BBOX * total_bbox + total_conf + gap
      rate = tm / nboxes
      o = jnp.where(flat_s == 0, loss,
          jnp.where(flat_s == 1, total_bbox,
          jnp.where(flat_s == 2, total_conf,
          jnp.where(flat_s == 3, gap,
          jnp.where(flat_s == 4, rate, 0.0)))))
      out_ref[...] = o

  return _body


def kernel(images, bboxes, preds):
  B, N, _ = preds.shape
  G = bboxes.shape[1]
  C = 128
  NPAD = ((N + 1023) // 1024) * 1024
  R = NPAD // C
  BPG = 4 if B % 4 == 0 else 1
  preds_p = jnp.pad(preds, ((0, 0), (0, NPAD - N), (0, 0)))
  pch = preds_p.transpose(0, 2, 1).reshape(B, 5, R, C)

  out = pl.pallas_call(
      _make_body(N, B, BPG),
      grid=(B // BPG,),
      in_specs=[
          pl.BlockSpec((BPG, 5, R, C), lambda i: (i, 0, 0, 0)),
          pl.BlockSpec((BPG, G, 4), lambda i: (i, 0, 0),
                       memory_space=pltpu.SMEM),
      ],
      out_specs=pl.BlockSpec((8, 128), lambda i: (0, 0)),
      out_shape=jax.ShapeDtypeStruct((8, 128), jnp.float32),
      scratch_shapes=(
          [pltpu.VMEM((G, R, C), jnp.float32) for _ in range(BPG)]
          + [pltpu.SMEM((128,), jnp.int32) for _ in range(BPG)]
          + [pltpu.SMEM((3,), jnp.float32)]
      ),
  )(pch, bboxes)
  return (out[0, 0], out[0, 1], out[0, 2], out[0, 3], out[0, 4])
